# baseline (speedup 1.0000x reference)
import functools

import jax
import jax.numpy as jnp
from jax import lax
from jax.experimental import pallas as pl
from jax.experimental.pallas import tpu as pltpu
from jax.experimental.pallas import tpu_sc as plsc

N_ATOMS = 262144
D_OUT = 366
D_EMB = 87
D_TOP = 88
T_STRIDE = 88
CA = 128
LANES = 16
TC_COLS = 8192


@functools.lru_cache(maxsize=1)
def _build_sc():
    info = plsc.get_sparse_core_info()
    nw = info.num_cores * info.num_subcores
    atoms_per_w = N_ATOMS // nw
    n_blocks = atoms_per_w // CA
    n_pairs = n_blocks // 2
    groups = CA // LANES

    mesh = plsc.VectorSubcoreMesh(core_axis_name="c", subcore_axis_name="s")

    @functools.partial(
        pl.kernel,
        mesh=mesh,
        compiler_params=pltpu.CompilerParams(needs_layout_passes=False),
        out_type=jax.ShapeDtypeStruct((D_OUT, N_ATOMS), jnp.float32),
        scratch_types=[
            pltpu.VMEM((D_EMB * T_STRIDE,), jnp.float32),
            pltpu.VMEM((atoms_per_w,), jnp.int32),
            pltpu.VMEM((D_TOP, CA), jnp.float32),
            pltpu.VMEM((D_TOP, CA), jnp.float32),
            pltpu.SemaphoreType.DMA,
            pltpu.SemaphoreType.DMA,
        ],
    )
    def k(z_hbm, tableT_hbm, out_hbm, tab_v, zslab, buf0, buf1, sem0, sem1):
        wid = lax.axis_index("s") * info.num_cores + lax.axis_index("c")
        col0 = wid * atoms_per_w

        pltpu.sync_copy(tableT_hbm, tab_v)
        pltpu.sync_copy(z_hbm.at[pl.ds(col0, atoms_per_w)], zslab)

        zero16 = jnp.zeros((LANES,), jnp.float32)
        for g in range(groups):
            buf0[D_EMB, pl.ds(g * LANES, LANES)] = zero16
            buf1[D_EMB, pl.ds(g * LANES, LANES)] = zero16

        def fill(buf, i):
            for g in range(groups):
                zv = zslab[pl.ds(i * CA + g * LANES, LANES)]

                def drow(d, _):
                    buf[d, pl.ds(g * LANES, LANES)] = plsc.load_gather(
                        tab_v, [d * T_STRIDE + zv]
                    )
                    return _

                lax.fori_loop(0, D_EMB, drow, 0, unroll=8)

        def start(buf, i, sem):
            return pltpu.async_copy(
                buf,
                out_hbm.at[pl.ds(0, D_TOP), pl.ds(col0 + i * CA, CA)],
                sem,
            )

        def drain(buf, sem):
            pltpu.make_async_copy(
                buf, out_hbm.at[pl.ds(0, D_TOP), pl.ds(col0, CA)], sem
            ).wait()

        def body(j, _):
            @pl.when(j > 0)
            def _w0():
                drain(buf0, sem0)

            fill(buf0, 2 * j)
            start(buf0, 2 * j, sem0)

            @pl.when(j > 0)
            def _w1():
                drain(buf1, sem1)

            fill(buf1, 2 * j + 1)
            start(buf1, 2 * j + 1, sem1)
            return _

        lax.fori_loop(0, n_pairs, body, 0)
        drain(buf0, sem0)
        drain(buf1, sem1)

    return k


@functools.lru_cache(maxsize=1)
def _build_tc_zero():
    n_row_blocks = (D_OUT - D_TOP + 7) // 8

    def body(_, o_ref):
        o_ref[...] = jnp.zeros_like(o_ref)

    return pl.pallas_call(
        body,
        grid=(n_row_blocks, N_ATOMS // TC_COLS),
        in_specs=[pl.BlockSpec(memory_space=pl.ANY)],
        out_specs=pl.BlockSpec(
            (8, TC_COLS), lambda r, c: (r + D_TOP // 8, c)
        ),
        out_shape=jax.ShapeDtypeStruct((D_OUT, N_ATOMS), jnp.float32),
        input_output_aliases={0: 0},
    )


def kernel(Z, emb_table):
    z_flat = Z.reshape(-1)
    tableT = (
        jnp.zeros((D_EMB, T_STRIDE), jnp.float32)
        .at[:, :D_EMB]
        .set(emb_table.T)
    )
    top_filled = _build_sc()(z_flat, tableT.reshape(-1))
    out_t = _build_tc_zero()(top_filled)
    return out_t.T


# speedup vs baseline: 4.3220x; 4.3220x over previous
import functools

import jax
import jax.numpy as jnp
from jax import lax
from jax.experimental import pallas as pl
from jax.experimental.pallas import tpu as pltpu
from jax.experimental.pallas import tpu_sc as plsc

N_ATOMS = 262144
D_OUT = 366
D_EMB = 87
T_STRIDE = 88
CA = 128
LANES = 16


@functools.lru_cache(maxsize=1)
def _build():
    info = plsc.get_sparse_core_info()
    nw = info.num_cores * info.num_subcores
    atoms_per_w = N_ATOMS // nw
    n_blocks = atoms_per_w // CA
    n_pairs = n_blocks // 2
    groups = CA // LANES

    mesh = plsc.VectorSubcoreMesh(core_axis_name="c", subcore_axis_name="s")

    @functools.partial(
        pl.kernel,
        mesh=mesh,
        compiler_params=pltpu.CompilerParams(needs_layout_passes=False),
        out_type=jax.ShapeDtypeStruct((D_OUT, N_ATOMS), jnp.float32),
        scratch_types=[
            pltpu.VMEM((D_EMB * T_STRIDE,), jnp.float32),
            pltpu.VMEM((atoms_per_w,), jnp.int32),
            pltpu.VMEM((D_OUT, CA), jnp.float32),
            pltpu.VMEM((D_OUT, CA), jnp.float32),
            pltpu.SemaphoreType.DMA,
            pltpu.SemaphoreType.DMA,
        ],
    )
    def k(z_hbm, tableT_hbm, out_hbm, tab_v, zslab, buf0, buf1, sem0, sem1):
        wid = lax.axis_index("s") * info.num_cores + lax.axis_index("c")
        col0 = wid * atoms_per_w

        pltpu.sync_copy(tableT_hbm, tab_v)
        pltpu.sync_copy(z_hbm.at[pl.ds(col0, atoms_per_w)], zslab)

        zero16 = jnp.zeros((LANES,), jnp.float32)

        def zrow(d, _):
            for g in range(groups):
                buf0[d, pl.ds(g * LANES, LANES)] = zero16
                buf1[d, pl.ds(g * LANES, LANES)] = zero16
            return _

        lax.fori_loop(0, D_OUT, zrow, 0, unroll=2)

        def fill(buf, i):
            zvs = [
                zslab[pl.ds(i * CA + g * LANES, LANES)] for g in range(groups)
            ]

            def drow(d, _):
                base = d * T_STRIDE
                xs = [
                    plsc.load_gather(tab_v, [base + zvs[g]])
                    for g in range(groups)
                ]
                for g in range(groups):
                    buf[d, pl.ds(g * LANES, LANES)] = xs[g]
                return _

            lax.fori_loop(0, D_EMB, drow, 0, unroll=2)

        def start(buf, i, sem):
            return pltpu.async_copy(
                buf, out_hbm.at[:, pl.ds(col0 + i * CA, CA)], sem
            )

        def drain(buf, sem):
            pltpu.make_async_copy(
                buf, out_hbm.at[:, pl.ds(col0, CA)], sem
            ).wait()

        def body(j, _):
            @pl.when(j > 0)
            def _w0():
                drain(buf0, sem0)

            fill(buf0, 2 * j)
            start(buf0, 2 * j, sem0)

            @pl.when(j > 0)
            def _w1():
                drain(buf1, sem1)

            fill(buf1, 2 * j + 1)
            start(buf1, 2 * j + 1, sem1)
            return _

        lax.fori_loop(0, n_pairs, body, 0)
        drain(buf0, sem0)
        drain(buf1, sem1)

    return k


def kernel(Z, emb_table):
    z_flat = Z.reshape(-1)
    tableT = (
        jnp.zeros((D_EMB, T_STRIDE), jnp.float32)
        .at[:, :D_EMB]
        .set(emb_table.T)
    )
    out_t = _build()(z_flat, tableT.reshape(-1))
    return out_t.T
